# baseline (speedup 1.0000x reference)
import functools

import jax
import jax.numpy as jnp
from jax import lax
from jax.experimental import pallas as pl
from jax.experimental.pallas import tpu as pltpu
from jax.experimental.pallas import tpu_sc as plsc

f32 = jnp.float32
i32 = jnp.int32

N_NODES = 10000
NP = 10240
E_TOT = 330000
CHUNK = 1024
EP = 344064
D = 128
BN = 512
NEG = -1e30


def _tc_project(xT, wT, sbd, dbd, bias_mat):
    nblk = NP // BN
    use_bias = bias_mat is not None

    def body(x_ref, w_ref, s_ref, d_ref, b_ref, h_ref, as_ref, ad_ref,
             ms_ref, md_ref):
        j = pl.program_id(0)
        g = x_ref[...]
        if use_bias:
            g = jnp.maximum(g + b_ref[...][:, 0:1], 0.0)
        h = jnp.dot(w_ref[...], g, preferred_element_type=f32)
        h_ref[...] = h
        asb = jnp.dot(s_ref[...], h, preferred_element_type=f32)
        adb = jnp.dot(d_ref[...], h, preferred_element_type=f32)
        as_ref[...] = asb
        ad_ref[...] = adb

        @pl.when(j == 0)
        def _():
            neg = jnp.full((8, D), NEG, f32)
            ms_ref[...] = neg
            md_ref[...] = neg

        ms_ref[...] = jnp.maximum(ms_ref[...],
                                  jnp.max(asb, axis=1, keepdims=True))
        md_ref[...] = jnp.maximum(md_ref[...],
                                  jnp.max(adb, axis=1, keepdims=True))

    if bias_mat is None:
        bias_mat = jnp.zeros((D, D), f32)
    return pl.pallas_call(
        body,
        grid=(nblk,),
        in_specs=[
            pl.BlockSpec((D, BN), lambda j: (0, j)),
            pl.BlockSpec((D, D), lambda j: (0, 0)),
            pl.BlockSpec((8, D), lambda j: (0, 0)),
            pl.BlockSpec((8, D), lambda j: (0, 0)),
            pl.BlockSpec((D, D), lambda j: (0, 0)),
        ],
        out_specs=[
            pl.BlockSpec((D, BN), lambda j: (0, j)),
            pl.BlockSpec((8, BN), lambda j: (0, j)),
            pl.BlockSpec((8, BN), lambda j: (0, j)),
            pl.BlockSpec((8, D), lambda j: (0, 0)),
            pl.BlockSpec((8, D), lambda j: (0, 0)),
        ],
        out_shape=[
            jax.ShapeDtypeStruct((D, NP), f32),
            jax.ShapeDtypeStruct((8, NP), f32),
            jax.ShapeDtypeStruct((8, NP), f32),
            jax.ShapeDtypeStruct((8, D), f32),
            jax.ShapeDtypeStruct((8, D), f32),
        ],
    )(xT, wT, sbd, dbd, bias_mat)


def _sc_gat_layer(groups, src, dst, asT, adT, hT, mb):
    qpg = 16 // groups
    ept_a = EP // qpg
    nck_a = ept_a // CHUNK
    nck_b = EP // CHUNK
    nv = CHUNK // 16
    nvz = NP // 16
    cpg = 64 // groups
    ST = NP // qpg
    U = 8

    @functools.partial(
        pl.kernel,
        out_type=jax.ShapeDtypeStruct((D * NP,), f32),
        mesh=plsc.VectorSubcoreMesh(core_axis_name="c", subcore_axis_name="s"),
        compiler_params=pltpu.CompilerParams(needs_layout_passes=False),
        scratch_types=[
            pltpu.VMEM((4 * NP,), f32),
            pltpu.VMEM((4 * NP,), f32),
            pltpu.VMEM((NP,), f32),
            pltpu.VMEM((NP,), f32),
            pltpu.VMEM((NP,), f32),
            pltpu.VMEM((ST,), f32),
            pltpu.VMEM((2 * CHUNK,), i32),
            pltpu.VMEM((2 * CHUNK,), i32),
            pltpu.VMEM((128,), f32),
            pltpu.VMEM_SHARED((16 * NP,), f32),
            pltpu.SemaphoreType.DMA((2,)),
            pltpu.SemaphoreType.DMA,
        ],
    )
    def sck(src_h, dst_h, as_h, ad_h, h_h, m_h, out_h,
            htab, otab, astab, adtab, denom, mrgb, srcb, dstb, mv,
            parts_sh, esem, hsem):
        cid = lax.axis_index("c")
        sid = lax.axis_index("s")
        hl = sid // qpg
        q = lax.rem(sid, qpg)
        trow = cid * groups + hl if groups > 1 else 0

        pltpu.sync_copy(as_h.at[pl.ds(pl.multiple_of(trow * NP, 128), NP)],
                        astab)
        pltpu.sync_copy(ad_h.at[pl.ds(pl.multiple_of(trow * NP, 128), NP)],
                        adtab)
        pltpu.sync_copy(m_h.at[pl.ds(pl.multiple_of(trow * 128, 128), 128)],
                        mv)
        astab[pl.ds(N_NODES, 16)] = jnp.full((16,), NEG, f32)
        adtab[pl.ds(N_NODES, 16)] = jnp.full((16,), NEG, f32)
        msub = mv[pl.ds(0, 16)]

        ch0 = cid * 64 + hl * cpg + q * 4
        hcp = pltpu.make_async_copy(
            h_h.at[pl.ds(pl.multiple_of(ch0 * NP, 128), 4 * NP)],
            htab, hsem)
        hcp.start()

        def zo(i, _):
            otab[pl.ds(i * 16, 16)] = jnp.zeros((16,), f32)
            return 0

        lax.fori_loop(0, 4 * nvz, zo, 0)

        def zv(i, _):
            denom[pl.ds(i * 16, 16)] = jnp.zeros((16,), f32)
            return 0

        lax.fori_loop(0, nvz, zv, 0)

        def edge_loop(base, nck, vec_body):
            def issue(k, b):
                off = pl.multiple_of(base + k * CHUNK, 128)
                bb = pl.multiple_of(b * CHUNK, 128)
                pltpu.async_copy(src_h.at[pl.ds(off, CHUNK)],
                                 srcb.at[pl.ds(bb, CHUNK)], esem.at[b])
                pltpu.async_copy(dst_h.at[pl.ds(off, CHUNK)],
                                 dstb.at[pl.ds(bb, CHUNK)], esem.at[b])

            issue(0, 0)
            issue(1, 1)

            def body(k, _):
                b = lax.rem(k, 2)
                bb = pl.multiple_of(b * CHUNK, 128)
                pltpu.make_async_copy(src_h.at[pl.ds(0, CHUNK)],
                                      srcb.at[pl.ds(bb, CHUNK)],
                                      esem.at[b]).wait()
                pltpu.make_async_copy(dst_h.at[pl.ds(0, CHUNK)],
                                      dstb.at[pl.ds(bb, CHUNK)],
                                      esem.at[b]).wait()

                @plsc.parallel_loop(0, nv, step=1, unroll=U)
                def _vloop(i):
                    vec_body(bb, i)

                @pl.when(k + 2 < nck)
                def _():
                    issue(k + 2, b)

                return 0

            lax.fori_loop(0, nck, body, 0)

        def vec_a(bb, iv):
            s16 = srcb[pl.ds(bb + iv * 16, 16)]
            d16 = dstb[pl.ds(bb + iv * 16, 16)]
            av = plsc.load_gather(astab, [s16])
            bv = plsc.load_gather(adtab, [d16])
            al = av + bv
            al = jnp.where(al >= 0.0, al, al * 0.2)
            ex = jnp.exp(al - msub)
            plsc.addupdate_scatter(denom, [d16], ex)

        with jax.named_scope("sc_passA"):
            edge_loop(q * ept_a, nck_a, vec_a)

        pltpu.sync_copy(denom,
                        parts_sh.at[pl.ds(pl.multiple_of(sid * NP, 128), NP)])
        plsc.subcore_barrier()

        so = pl.multiple_of(q * ST, 128)
        row0 = pl.multiple_of(hl * qpg * NP, 128)
        pltpu.sync_copy(parts_sh.at[pl.ds(row0 + so, ST)],
                        denom.at[pl.ds(so, ST)])

        def merge_j(j, _):
            pltpu.sync_copy(
                parts_sh.at[pl.ds(
                    pl.multiple_of((hl * qpg + j) * NP, 128) + so, ST)],
                mrgb)

            def addv(i, _):
                denom[pl.ds(so + i * 16, 16)] = (
                    denom[pl.ds(so + i * 16, 16)] + mrgb[pl.ds(i * 16, 16)])
                return 0

            lax.fori_loop(0, ST // 16, addv, 0)
            return 0

        lax.fori_loop(1, qpg, merge_j, 0)

        def rdv(i, _):
            dv = denom[pl.ds(so + i * 16, 16)]
            denom[pl.ds(so + i * 16, 16)] = 1.0 / (dv + 1e-16)
            return 0

        lax.fori_loop(0, ST // 16, rdv, 0)

        plsc.subcore_barrier()
        pltpu.sync_copy(denom.at[pl.ds(so, ST)],
                        parts_sh.at[pl.ds(row0 + so, ST)])
        plsc.subcore_barrier()
        pltpu.sync_copy(parts_sh.at[pl.ds(row0, NP)], denom)
        hcp.wait()

        def vec_b(bb, iv):
            s16 = srcb[pl.ds(bb + iv * 16, 16)]
            d16 = dstb[pl.ds(bb + iv * 16, 16)]
            av = plsc.load_gather(astab, [s16])
            bv = plsc.load_gather(adtab, [d16])
            al = av + bv
            al = jnp.where(al >= 0.0, al, al * 0.2)
            ex = jnp.exp(al - msub)
            rd = plsc.load_gather(denom, [d16])
            coef = ex * rd
            for r in range(4):
                hv = plsc.load_gather(htab, [s16 + r * NP])
                plsc.addupdate_scatter(otab, [d16 + r * NP], hv * coef)

        with jax.named_scope("sc_passB"):
            edge_loop(0, nck_b, vec_b)

        pltpu.sync_copy(otab,
                        out_h.at[pl.ds(pl.multiple_of(ch0 * NP, 128),
                                       4 * NP)])

    return sck(src, dst, asT, adT, hT, mb)


def kernel(x, edge_index, W1, att_src1, att_dst1, b1,
           W2, att_src2, att_dst2, b2):
    ei = edge_index.astype(i32)
    loop = jnp.arange(N_NODES, dtype=i32)
    padv = jnp.full((EP - E_TOT,), N_NODES, i32)
    src = jnp.concatenate([ei[0], loop, padv])
    dst = jnp.concatenate([ei[1], loop, padv])

    xT = jnp.zeros((D, NP), f32).at[:, :N_NODES].set(x.T)

    heads, cph = 8, 16
    cols = (jnp.arange(heads) * cph)[:, None] + jnp.arange(cph)[None, :]
    rows = jnp.arange(heads)[:, None]
    sbd1 = jnp.zeros((heads, D), f32).at[rows, cols].set(
        att_src1.reshape(heads, cph))
    dbd1 = jnp.zeros((heads, D), f32).at[rows, cols].set(
        att_dst1.reshape(heads, cph))

    h1T, as1, ad1, ms1, md1 = _tc_project(xT, W1.T, sbd1, dbd1, None)
    m1 = (ms1 + md1).reshape(-1)
    out1T = _sc_gat_layer(4, src, dst, as1.reshape(-1), ad1.reshape(-1),
                          h1T.reshape(-1), m1).reshape(D, NP)

    sbd2 = jnp.zeros((8, D), f32).at[0].set(att_src2.reshape(D))
    dbd2 = jnp.zeros((8, D), f32).at[0].set(att_dst2.reshape(D))
    b1m = jnp.broadcast_to(b1[:, None], (D, D))
    h2T, as2, ad2, ms2, md2 = _tc_project(out1T, W2.T, sbd2, dbd2, b1m)
    m2 = (ms2 + md2).reshape(-1)
    out2T = _sc_gat_layer(1, src, dst, as2.reshape(-1), ad2.reshape(-1),
                          h2T.reshape(-1), m2).reshape(D, NP)

    return out2T[:, :N_NODES].T + b2[None, :]


# speedup vs baseline: 1.3493x; 1.3493x over previous
import functools

import jax
import jax.numpy as jnp
from jax import lax
from jax.experimental import pallas as pl
from jax.experimental.pallas import tpu as pltpu
from jax.experimental.pallas import tpu_sc as plsc

f32 = jnp.float32
i32 = jnp.int32

N_NODES = 10000
NP = 10240
E_TOT = 330000
CHUNK = 1024
EP = 344064
D = 128
BN = 512
NEG = -1e30


def _tc_project(xT, wT, sbd, dbd, bias_mat):
    nblk = NP // BN
    use_bias = bias_mat is not None

    def body(x_ref, w_ref, s_ref, d_ref, b_ref, h_ref, as_ref, ad_ref,
             ms_ref, md_ref):
        j = pl.program_id(0)
        g = x_ref[...]
        if use_bias:
            g = jnp.maximum(g + b_ref[...][:, 0:1], 0.0)
        h = jnp.dot(w_ref[...], g, preferred_element_type=f32)
        h_ref[...] = h
        asb = jnp.dot(s_ref[...], h, preferred_element_type=f32)
        adb = jnp.dot(d_ref[...], h, preferred_element_type=f32)
        as_ref[...] = asb
        ad_ref[...] = adb

        @pl.when(j == 0)
        def _():
            neg = jnp.full((8, D), NEG, f32)
            ms_ref[...] = neg
            md_ref[...] = neg

        ms_ref[...] = jnp.maximum(ms_ref[...],
                                  jnp.max(asb, axis=1, keepdims=True))
        md_ref[...] = jnp.maximum(md_ref[...],
                                  jnp.max(adb, axis=1, keepdims=True))

    if bias_mat is None:
        bias_mat = jnp.zeros((D, D), f32)
    return pl.pallas_call(
        body,
        grid=(nblk,),
        in_specs=[
            pl.BlockSpec((D, BN), lambda j: (0, j)),
            pl.BlockSpec((D, D), lambda j: (0, 0)),
            pl.BlockSpec((8, D), lambda j: (0, 0)),
            pl.BlockSpec((8, D), lambda j: (0, 0)),
            pl.BlockSpec((D, D), lambda j: (0, 0)),
        ],
        out_specs=[
            pl.BlockSpec((D, BN), lambda j: (0, j)),
            pl.BlockSpec((8, BN), lambda j: (0, j)),
            pl.BlockSpec((8, BN), lambda j: (0, j)),
            pl.BlockSpec((8, D), lambda j: (0, 0)),
            pl.BlockSpec((8, D), lambda j: (0, 0)),
        ],
        out_shape=[
            jax.ShapeDtypeStruct((D, NP), f32),
            jax.ShapeDtypeStruct((8, NP), f32),
            jax.ShapeDtypeStruct((8, NP), f32),
            jax.ShapeDtypeStruct((8, D), f32),
            jax.ShapeDtypeStruct((8, D), f32),
        ],
    )(xT, wT, sbd, dbd, bias_mat)


def _sc_gat_layer(groups, src, dst, asT, adT, hT, mb):
    qpg = 16 // groups
    ept_a = EP // qpg
    nck_a = ept_a // CHUNK
    nck_b = EP // CHUNK
    nv = CHUNK // 16
    nvz = NP // 16
    cpg = 64 // groups
    ST = NP // qpg
    U = 2

    @functools.partial(
        pl.kernel,
        out_type=jax.ShapeDtypeStruct((D * NP,), f32),
        mesh=plsc.VectorSubcoreMesh(core_axis_name="c", subcore_axis_name="s"),
        compiler_params=pltpu.CompilerParams(needs_layout_passes=False),
        scratch_types=[
            pltpu.VMEM((4 * NP,), f32),
            pltpu.VMEM((4 * NP,), f32),
            pltpu.VMEM((NP,), f32),
            pltpu.VMEM((NP,), f32),
            pltpu.VMEM((NP,), f32),
            pltpu.VMEM((ST,), f32),
            pltpu.VMEM((2 * CHUNK,), i32),
            pltpu.VMEM((2 * CHUNK,), i32),
            pltpu.VMEM((128,), f32),
            pltpu.VMEM_SHARED((16 * NP,), f32),
            pltpu.SemaphoreType.DMA((2,)),
            pltpu.SemaphoreType.DMA,
        ],
    )
    def sck(src_h, dst_h, as_h, ad_h, h_h, m_h, out_h,
            htab, otab, astab, adtab, denom, mrgb, srcb, dstb, mv,
            parts_sh, esem, hsem):
        cid = lax.axis_index("c")
        sid = lax.axis_index("s")
        hl = sid // qpg
        q = lax.rem(sid, qpg)
        trow = cid * groups + hl if groups > 1 else 0

        pltpu.sync_copy(as_h.at[pl.ds(pl.multiple_of(trow * NP, 128), NP)],
                        astab)
        pltpu.sync_copy(ad_h.at[pl.ds(pl.multiple_of(trow * NP, 128), NP)],
                        adtab)
        pltpu.sync_copy(m_h.at[pl.ds(pl.multiple_of(trow * 128, 128), 128)],
                        mv)
        astab[pl.ds(N_NODES, 16)] = jnp.full((16,), NEG, f32)
        adtab[pl.ds(N_NODES, 16)] = jnp.full((16,), NEG, f32)
        msub = mv[pl.ds(0, 16)]

        ch0 = cid * 64 + hl * cpg + q * 4
        hcp = pltpu.make_async_copy(
            h_h.at[pl.ds(pl.multiple_of(ch0 * NP, 128), 4 * NP)],
            htab, hsem)
        hcp.start()

        def zo(i, _):
            otab[pl.ds(i * 16, 16)] = jnp.zeros((16,), f32)
            return 0

        lax.fori_loop(0, 4 * nvz, zo, 0)

        def zv(i, _):
            denom[pl.ds(i * 16, 16)] = jnp.zeros((16,), f32)
            return 0

        lax.fori_loop(0, nvz, zv, 0)

        def edge_loop(base, nck, vec_body):
            def issue(k, b):
                off = pl.multiple_of(base + k * CHUNK, 128)
                bb = pl.multiple_of(b * CHUNK, 128)
                pltpu.async_copy(src_h.at[pl.ds(off, CHUNK)],
                                 srcb.at[pl.ds(bb, CHUNK)], esem.at[b])
                pltpu.async_copy(dst_h.at[pl.ds(off, CHUNK)],
                                 dstb.at[pl.ds(bb, CHUNK)], esem.at[b])

            issue(0, 0)
            issue(1, 1)

            def body(k, _):
                b = lax.rem(k, 2)
                bb = pl.multiple_of(b * CHUNK, 128)
                pltpu.make_async_copy(src_h.at[pl.ds(0, CHUNK)],
                                      srcb.at[pl.ds(bb, CHUNK)],
                                      esem.at[b]).wait()
                pltpu.make_async_copy(dst_h.at[pl.ds(0, CHUNK)],
                                      dstb.at[pl.ds(bb, CHUNK)],
                                      esem.at[b]).wait()

                @plsc.parallel_loop(0, nv, step=1, unroll=U)
                def _vloop(i):
                    vec_body(bb, i)

                @pl.when(k + 2 < nck)
                def _():
                    issue(k + 2, b)

                return 0

            lax.fori_loop(0, nck, body, 0)

        def vec_a(bb, iv):
            s16 = srcb[pl.ds(bb + iv * 16, 16)]
            d16 = dstb[pl.ds(bb + iv * 16, 16)]
            av = plsc.load_gather(astab, [s16])
            bv = plsc.load_gather(adtab, [d16])
            al = av + bv
            al = jnp.where(al >= 0.0, al, al * 0.2)
            ex = jnp.exp(al - msub)
            plsc.addupdate_scatter(denom, [d16], ex)

        with jax.named_scope("sc_passA"):
            edge_loop(q * ept_a, nck_a, vec_a)

        pltpu.sync_copy(denom,
                        parts_sh.at[pl.ds(pl.multiple_of(sid * NP, 128), NP)])
        plsc.subcore_barrier()

        so = pl.multiple_of(q * ST, 128)
        row0 = pl.multiple_of(hl * qpg * NP, 128)
        pltpu.sync_copy(parts_sh.at[pl.ds(row0 + so, ST)],
                        denom.at[pl.ds(so, ST)])

        def merge_j(j, _):
            pltpu.sync_copy(
                parts_sh.at[pl.ds(
                    pl.multiple_of((hl * qpg + j) * NP, 128) + so, ST)],
                mrgb)

            def addv(i, _):
                denom[pl.ds(so + i * 16, 16)] = (
                    denom[pl.ds(so + i * 16, 16)] + mrgb[pl.ds(i * 16, 16)])
                return 0

            lax.fori_loop(0, ST // 16, addv, 0)
            return 0

        lax.fori_loop(1, qpg, merge_j, 0)

        def rdv(i, _):
            dv = denom[pl.ds(so + i * 16, 16)]
            denom[pl.ds(so + i * 16, 16)] = 1.0 / (dv + 1e-16)
            return 0

        lax.fori_loop(0, ST // 16, rdv, 0)

        plsc.subcore_barrier()
        pltpu.sync_copy(denom.at[pl.ds(so, ST)],
                        parts_sh.at[pl.ds(row0 + so, ST)])
        plsc.subcore_barrier()
        pltpu.sync_copy(parts_sh.at[pl.ds(row0, NP)], denom)
        hcp.wait()

        def vec_b(bb, iv):
            s16 = srcb[pl.ds(bb + iv * 16, 16)]
            d16 = dstb[pl.ds(bb + iv * 16, 16)]
            av = plsc.load_gather(astab, [s16])
            bv = plsc.load_gather(adtab, [d16])
            al = av + bv
            al = jnp.where(al >= 0.0, al, al * 0.2)
            ex = jnp.exp(al - msub)
            rd = plsc.load_gather(denom, [d16])
            coef = ex * rd
            for r in range(4):
                hv = plsc.load_gather(htab, [s16 + r * NP])
                plsc.addupdate_scatter(otab, [d16 + r * NP], hv * coef)

        with jax.named_scope("sc_passB"):
            edge_loop(0, nck_b, vec_b)

        pltpu.sync_copy(otab,
                        out_h.at[pl.ds(pl.multiple_of(ch0 * NP, 128),
                                       4 * NP)])

    return sck(src, dst, asT, adT, hT, mb)


def kernel(x, edge_index, W1, att_src1, att_dst1, b1,
           W2, att_src2, att_dst2, b2):
    ei = edge_index.astype(i32)
    loop = jnp.arange(N_NODES, dtype=i32)
    padv = jnp.full((EP - E_TOT,), N_NODES, i32)
    src = jnp.concatenate([ei[0], loop, padv])
    dst = jnp.concatenate([ei[1], loop, padv])

    xT = jnp.zeros((D, NP), f32).at[:, :N_NODES].set(x.T)

    heads, cph = 8, 16
    cols = (jnp.arange(heads) * cph)[:, None] + jnp.arange(cph)[None, :]
    rows = jnp.arange(heads)[:, None]
    sbd1 = jnp.zeros((heads, D), f32).at[rows, cols].set(
        att_src1.reshape(heads, cph))
    dbd1 = jnp.zeros((heads, D), f32).at[rows, cols].set(
        att_dst1.reshape(heads, cph))

    h1T, as1, ad1, ms1, md1 = _tc_project(xT, W1.T, sbd1, dbd1, None)
    m1 = (ms1 + md1).reshape(-1)
    out1T = _sc_gat_layer(4, src, dst, as1.reshape(-1), ad1.reshape(-1),
                          h1T.reshape(-1), m1).reshape(D, NP)

    sbd2 = jnp.zeros((8, D), f32).at[0].set(att_src2.reshape(D))
    dbd2 = jnp.zeros((8, D), f32).at[0].set(att_dst2.reshape(D))
    b1m = jnp.broadcast_to(b1[:, None], (D, D))
    h2T, as2, ad2, ms2, md2 = _tc_project(out1T, W2.T, sbd2, dbd2, b1m)
    m2 = (ms2 + md2).reshape(-1)
    out2T = _sc_gat_layer(1, src, dst, as2.reshape(-1), ad2.reshape(-1),
                          h2T.reshape(-1), m2).reshape(D, NP)

    return out2T[:, :N_NODES].T + b2[None, :]


# speedup vs baseline: 1.3692x; 1.0148x over previous
import functools

import jax
import jax.numpy as jnp
from jax import lax
from jax.experimental import pallas as pl
from jax.experimental.pallas import tpu as pltpu
from jax.experimental.pallas import tpu_sc as plsc

f32 = jnp.float32
i32 = jnp.int32

N_NODES = 10000
NP = 10240
E_TOT = 330000
CHUNK = 1024
EP = 344064
D = 128
BN = 512
NEG = -1e30


def _tc_project(xT, wT, sbd, dbd, bias_mat):
    nblk = NP // BN
    use_bias = bias_mat is not None

    def body(x_ref, w_ref, s_ref, d_ref, b_ref, h_ref, as_ref, ad_ref,
             ms_ref, md_ref):
        j = pl.program_id(0)
        g = x_ref[...]
        if use_bias:
            g = jnp.maximum(g + b_ref[...][:, 0:1], 0.0)
        h = jnp.dot(w_ref[...], g, preferred_element_type=f32)
        h_ref[...] = h
        asb = jnp.dot(s_ref[...], h, preferred_element_type=f32)
        adb = jnp.dot(d_ref[...], h, preferred_element_type=f32)
        as_ref[...] = asb
        ad_ref[...] = adb

        @pl.when(j == 0)
        def _():
            neg = jnp.full((8, D), NEG, f32)
            ms_ref[...] = neg
            md_ref[...] = neg

        ms_ref[...] = jnp.maximum(ms_ref[...],
                                  jnp.max(asb, axis=1, keepdims=True))
        md_ref[...] = jnp.maximum(md_ref[...],
                                  jnp.max(adb, axis=1, keepdims=True))

    if bias_mat is None:
        bias_mat = jnp.zeros((D, D), f32)
    return pl.pallas_call(
        body,
        grid=(nblk,),
        in_specs=[
            pl.BlockSpec((D, BN), lambda j: (0, j)),
            pl.BlockSpec((D, D), lambda j: (0, 0)),
            pl.BlockSpec((8, D), lambda j: (0, 0)),
            pl.BlockSpec((8, D), lambda j: (0, 0)),
            pl.BlockSpec((D, D), lambda j: (0, 0)),
        ],
        out_specs=[
            pl.BlockSpec((D, BN), lambda j: (0, j)),
            pl.BlockSpec((8, BN), lambda j: (0, j)),
            pl.BlockSpec((8, BN), lambda j: (0, j)),
            pl.BlockSpec((8, D), lambda j: (0, 0)),
            pl.BlockSpec((8, D), lambda j: (0, 0)),
        ],
        out_shape=[
            jax.ShapeDtypeStruct((D, NP), f32),
            jax.ShapeDtypeStruct((8, NP), f32),
            jax.ShapeDtypeStruct((8, NP), f32),
            jax.ShapeDtypeStruct((8, D), f32),
            jax.ShapeDtypeStruct((8, D), f32),
        ],
    )(xT, wT, sbd, dbd, bias_mat)


def _sc_gat_layer(groups, src, dst, asT, adT, hT, mb):
    qpg = 16 // groups
    ept_a = EP // qpg
    nck_a = ept_a // CHUNK
    nck_b = EP // CHUNK
    nv = CHUNK // 16
    nvz = NP // 16
    cpg = 64 // groups
    ST = NP // qpg
    PS = min(ST, CHUNK)
    U = 2

    @functools.partial(
        pl.kernel,
        out_type=(jax.ShapeDtypeStruct((D * NP,), f32),
                  jax.ShapeDtypeStruct((2 * groups * EP,), f32)),
        mesh=plsc.VectorSubcoreMesh(core_axis_name="c", subcore_axis_name="s"),
        compiler_params=pltpu.CompilerParams(needs_layout_passes=False),
        scratch_types=[
            pltpu.VMEM((4 * NP,), f32),
            pltpu.VMEM((4 * NP,), f32),
            pltpu.VMEM((NP,), f32),
            pltpu.VMEM((NP,), f32),
            pltpu.VMEM((NP,), f32),
            pltpu.VMEM((2 * CHUNK,), i32),
            pltpu.VMEM((2 * CHUNK,), i32),
            pltpu.VMEM((2 * CHUNK,), f32),
            pltpu.VMEM((128,), f32),
            pltpu.VMEM_SHARED((16 * NP,), f32),
            pltpu.SemaphoreType.DMA((2,)),
            pltpu.SemaphoreType.DMA((2,)),
            pltpu.SemaphoreType.DMA,
        ],
    )
    def sck(src_h, dst_h, as_h, ad_h, h_h, m_h, out_h, ex_h,
            htab, otab, astab, adtab, denom, srcb, dstb, exb, mv,
            parts_sh, esem, csem, hsem):
        cid = lax.axis_index("c")
        sid = lax.axis_index("s")
        hl = sid // qpg
        q = lax.rem(sid, qpg)
        trow = cid * groups + hl if groups > 1 else 0
        exrow = pl.multiple_of((cid * groups + hl) * EP, 128)

        pltpu.sync_copy(as_h.at[pl.ds(pl.multiple_of(trow * NP, 128), NP)],
                        astab)
        pltpu.sync_copy(ad_h.at[pl.ds(pl.multiple_of(trow * NP, 128), NP)],
                        adtab)
        pltpu.sync_copy(m_h.at[pl.ds(pl.multiple_of(trow * 128, 128), 128)],
                        mv)
        astab[pl.ds(N_NODES, 16)] = jnp.full((16,), NEG, f32)
        adtab[pl.ds(N_NODES, 16)] = jnp.full((16,), NEG, f32)
        msub = mv[pl.ds(0, 16)]

        ch0 = cid * 64 + hl * cpg + q * 4
        hcp = pltpu.make_async_copy(
            h_h.at[pl.ds(pl.multiple_of(ch0 * NP, 128), 4 * NP)],
            htab, hsem)
        hcp.start()

        def zo(i, _):
            otab[pl.ds(i * 16, 16)] = jnp.zeros((16,), f32)
            return 0

        lax.fori_loop(0, 4 * nvz, zo, 0)

        def zv(i, _):
            denom[pl.ds(i * 16, 16)] = jnp.zeros((16,), f32)
            return 0

        lax.fori_loop(0, nvz, zv, 0)

        def edge_loop(base, nck, vec_body, ex_in=False, ex_out=False):
            def issue(k, b):
                off = pl.multiple_of(base + k * CHUNK, 128)
                bb = pl.multiple_of(b * CHUNK, 128)
                pltpu.async_copy(src_h.at[pl.ds(off, CHUNK)],
                                 srcb.at[pl.ds(bb, CHUNK)], esem.at[b])
                pltpu.async_copy(dst_h.at[pl.ds(off, CHUNK)],
                                 dstb.at[pl.ds(bb, CHUNK)], esem.at[b])
                if ex_in:
                    pltpu.async_copy(ex_h.at[pl.ds(exrow + off, CHUNK)],
                                     exb.at[pl.ds(bb, CHUNK)], esem.at[b])

            issue(0, 0)
            issue(1, 1)

            def body(k, _):
                b = lax.rem(k, 2)
                bb = pl.multiple_of(b * CHUNK, 128)
                pltpu.make_async_copy(src_h.at[pl.ds(0, CHUNK)],
                                      srcb.at[pl.ds(bb, CHUNK)],
                                      esem.at[b]).wait()
                pltpu.make_async_copy(dst_h.at[pl.ds(0, CHUNK)],
                                      dstb.at[pl.ds(bb, CHUNK)],
                                      esem.at[b]).wait()
                if ex_in:
                    pltpu.make_async_copy(src_h.at[pl.ds(0, CHUNK)],
                                          exb.at[pl.ds(bb, CHUNK)],
                                          esem.at[b]).wait()
                if ex_out:
                    @pl.when(k >= 2)
                    def _():
                        pltpu.make_async_copy(src_h.at[pl.ds(0, CHUNK)],
                                              exb.at[pl.ds(bb, CHUNK)],
                                              csem.at[b]).wait()

                @plsc.parallel_loop(0, nv, step=1, unroll=U)
                def _vloop(i):
                    vec_body(bb, i)

                if ex_out:
                    off = pl.multiple_of(base + k * CHUNK, 128)
                    pltpu.async_copy(exb.at[pl.ds(bb, CHUNK)],
                                     ex_h.at[pl.ds(exrow + off, CHUNK)],
                                     csem.at[b])

                @pl.when(k + 2 < nck)
                def _():
                    issue(k + 2, b)

                return 0

            lax.fori_loop(0, nck, body, 0)

            if ex_out:
                for b in range(2):
                    @pl.when(nck >= 2 - b)
                    def _():
                        pltpu.make_async_copy(
                            src_h.at[pl.ds(0, CHUNK)],
                            exb.at[pl.ds(pl.multiple_of(b * CHUNK, 128),
                                         CHUNK)],
                            csem.at[b]).wait()

        def vec_a(bb, iv):
            s16 = srcb[pl.ds(bb + iv * 16, 16)]
            d16 = dstb[pl.ds(bb + iv * 16, 16)]
            av = plsc.load_gather(astab, [s16])
            bv = plsc.load_gather(adtab, [d16])
            al = av + bv
            al = jnp.where(al >= 0.0, al, al * 0.2)
            ex = jnp.exp(al - msub)
            exb[pl.ds(bb + iv * 16, 16)] = ex
            plsc.addupdate_scatter(denom, [d16], ex)

        with jax.named_scope("sc_passA"):
            edge_loop(q * ept_a, nck_a, vec_a, ex_out=True)

        pltpu.sync_copy(denom,
                        parts_sh.at[pl.ds(pl.multiple_of(sid * NP, 128), NP)])
        plsc.subcore_barrier()

        so = pl.multiple_of(q * ST, 128)
        row0 = pl.multiple_of(hl * qpg * NP, 128)
        pltpu.sync_copy(parts_sh.at[pl.ds(row0 + so, ST)],
                        denom.at[pl.ds(so, ST)])

        def merge_j(j, _):
            def piece(p, _):
                po = pl.multiple_of(p * PS, 128)
                pltpu.sync_copy(
                    parts_sh.at[pl.ds(
                        pl.multiple_of((hl * qpg + j) * NP, 128) + so + po,
                        PS)],
                    exb.at[pl.ds(0, PS)])

                def addv(i, _):
                    denom[pl.ds(so + po + i * 16, 16)] = (
                        denom[pl.ds(so + po + i * 16, 16)]
                        + exb[pl.ds(i * 16, 16)])
                    return 0

                lax.fori_loop(0, PS // 16, addv, 0)
                return 0

            lax.fori_loop(0, ST // PS, piece, 0)
            return 0

        lax.fori_loop(1, qpg, merge_j, 0)

        def rdv(i, _):
            dv = denom[pl.ds(so + i * 16, 16)]
            denom[pl.ds(so + i * 16, 16)] = 1.0 / (dv + 1e-16)
            return 0

        lax.fori_loop(0, ST // 16, rdv, 0)

        plsc.subcore_barrier()
        pltpu.sync_copy(denom.at[pl.ds(so, ST)],
                        parts_sh.at[pl.ds(row0 + so, ST)])
        plsc.subcore_barrier()
        pltpu.sync_copy(parts_sh.at[pl.ds(row0, NP)], denom)
        hcp.wait()

        def vec_b(bb, iv):
            s16 = srcb[pl.ds(bb + iv * 16, 16)]
            d16 = dstb[pl.ds(bb + iv * 16, 16)]
            ex = exb[pl.ds(bb + iv * 16, 16)]
            rd = plsc.load_gather(denom, [d16])
            coef = ex * rd
            for r in range(4):
                hv = plsc.load_gather(htab, [s16 + r * NP])
                plsc.addupdate_scatter(otab, [d16 + r * NP], hv * coef)

        with jax.named_scope("sc_passB"):
            edge_loop(0, nck_b, vec_b, ex_in=True)

        pltpu.sync_copy(otab,
                        out_h.at[pl.ds(pl.multiple_of(ch0 * NP, 128),
                                       4 * NP)])

    return sck(src, dst, asT, adT, hT, mb)[0]


def kernel(x, edge_index, W1, att_src1, att_dst1, b1,
           W2, att_src2, att_dst2, b2):
    ei = edge_index.astype(i32)
    loop = jnp.arange(N_NODES, dtype=i32)
    padv = jnp.full((EP - E_TOT,), N_NODES, i32)
    src = jnp.concatenate([ei[0], loop, padv])
    dst = jnp.concatenate([ei[1], loop, padv])

    xT = jnp.zeros((D, NP), f32).at[:, :N_NODES].set(x.T)

    heads, cph = 8, 16
    cols = (jnp.arange(heads) * cph)[:, None] + jnp.arange(cph)[None, :]
    rows = jnp.arange(heads)[:, None]
    sbd1 = jnp.zeros((heads, D), f32).at[rows, cols].set(
        att_src1.reshape(heads, cph))
    dbd1 = jnp.zeros((heads, D), f32).at[rows, cols].set(
        att_dst1.reshape(heads, cph))

    h1T, as1, ad1, ms1, md1 = _tc_project(xT, W1.T, sbd1, dbd1, None)
    m1 = (ms1 + md1).reshape(-1)
    out1T = _sc_gat_layer(4, src, dst, as1.reshape(-1), ad1.reshape(-1),
                          h1T.reshape(-1), m1).reshape(D, NP)

    sbd2 = jnp.zeros((8, D), f32).at[0].set(att_src2.reshape(D))
    dbd2 = jnp.zeros((8, D), f32).at[0].set(att_dst2.reshape(D))
    b1m = jnp.broadcast_to(b1[:, None], (D, D))
    h2T, as2, ad2, ms2, md2 = _tc_project(out1T, W2.T, sbd2, dbd2, b1m)
    m2 = (ms2 + md2).reshape(-1)
    out2T = _sc_gat_layer(1, src, dst, as2.reshape(-1), ad2.reshape(-1),
                          h2T.reshape(-1), m2).reshape(D, NP)

    return out2T[:, :N_NODES].T + b2[None, :]
